# Initial kernel scaffold; baseline (speedup 1.0000x reference)
import jax
import jax.numpy as jnp
from jax.experimental import pallas as pl


def kernel(x_operation, batch_operation, tok_operation, Wl_operation, bl_operation, Wr_operation, br_operation, att_operation, bias_operation, x_machine, batch_machine, tok_machine, Wl_machine, bl_machine, Wr_machine, br_machine, att_machine, bias_machine, x_AGV, batch_AGV, tok_AGV, Wl_AGV, bl_AGV, Wr_AGV, br_AGV, att_AGV, bias_AGV, W1, b1, W2, b2):
    raise NotImplementedError("write your pallas kernel here")


# speedup vs baseline: 10.4756x; 10.4756x over previous
import functools

import jax
import jax.numpy as jnp
from jax.experimental import pallas as pl
from jax.experimental.pallas import tpu as pltpu

NUM_GRAPHS = 512
_CH = 256


def _gat_step(x_ref, b_ref, tok_ref, Wl_ref, bl_ref, Wr_ref, br_ref, att_ref,
              bias_ref, out_ref, acc_ref, den_ref, *, num_steps, tile):
    i = pl.program_id(0)

    @pl.when(i == 0)
    def _init():
        acc_ref[...] = jnp.zeros_like(acc_ref)
        den_ref[...] = jnp.zeros_like(den_ref)

    x = x_ref[...]
    xl = jnp.dot(x, Wl_ref[...],
                 preferred_element_type=jnp.float32) + bl_ref[...]
    r = jnp.dot(tok_ref[...], Wr_ref[...],
                preferred_element_type=jnp.float32) + br_ref[...]
    e = xl + r
    e = jnp.where(e >= 0.0, e, 0.2 * e)
    logit = jnp.sum(e * att_ref[...], axis=1, keepdims=True)
    ex = jnp.exp(logit)

    b = b_ref[...]
    onehot = (b == jax.lax.broadcasted_iota(jnp.int32, (tile, NUM_GRAPHS), 1))
    ow = onehot.astype(jnp.float32) * ex

    acc_ref[...] += jax.lax.dot_general(
        ow, xl, (((0,), (0,)), ((), ())),
        preferred_element_type=jnp.float32)
    den_ref[...] += jnp.sum(ow, axis=0, keepdims=True)

    @pl.when(i == num_steps - 1)
    def _finalize():
        den_col = jax.lax.transpose(den_ref[...], (1, 0))
        out_ref[...] = acc_ref[...] / (den_col + 1e-16) + bias_ref[...]


def _gat_global(x, batch, tok, Wl, bl, Wr, br, att, bias, *, tile):
    n = x.shape[0]
    assert n % tile == 0, (n, tile)
    num_steps = n // tile
    b2 = batch.reshape(n, 1)
    return pl.pallas_call(
        functools.partial(_gat_step, num_steps=num_steps, tile=tile),
        grid=(num_steps,),
        in_specs=[
            pl.BlockSpec((tile, _CH), lambda i: (i, 0)),
            pl.BlockSpec((tile, 1), lambda i: (i, 0)),
            pl.BlockSpec((1, _CH), lambda i: (0, 0)),
            pl.BlockSpec((_CH, _CH), lambda i: (0, 0)),
            pl.BlockSpec((1, _CH), lambda i: (0, 0)),
            pl.BlockSpec((_CH, _CH), lambda i: (0, 0)),
            pl.BlockSpec((1, _CH), lambda i: (0, 0)),
            pl.BlockSpec((1, _CH), lambda i: (0, 0)),
            pl.BlockSpec((1, _CH), lambda i: (0, 0)),
        ],
        out_specs=pl.BlockSpec((NUM_GRAPHS, _CH), lambda i: (0, 0)),
        out_shape=jax.ShapeDtypeStruct((NUM_GRAPHS, _CH), jnp.float32),
        scratch_shapes=[
            pltpu.VMEM((NUM_GRAPHS, _CH), jnp.float32),
            pltpu.VMEM((1, NUM_GRAPHS), jnp.float32),
        ],
        compiler_params=pltpu.CompilerParams(
            dimension_semantics=("arbitrary",)),
    )(x, b2, tok.reshape(1, _CH), Wl, bl.reshape(1, _CH), Wr,
      br.reshape(1, _CH), att.reshape(1, _CH), bias.reshape(1, _CH))


def _mlp_step(g0_ref, g1_ref, g2_ref, W1a_ref, W1b_ref, W1c_ref, b1_ref,
              W2_ref, b2_ref, out_ref):
    h = (jnp.dot(g0_ref[...], W1a_ref[...], preferred_element_type=jnp.float32)
         + jnp.dot(g1_ref[...], W1b_ref[...], preferred_element_type=jnp.float32)
         + jnp.dot(g2_ref[...], W1c_ref[...], preferred_element_type=jnp.float32)
         + b1_ref[...])
    h = jnp.where(h >= 0.0, h, 0.01 * h)
    out_ref[...] = jnp.dot(h, W2_ref[...],
                           preferred_element_type=jnp.float32) + b2_ref[...]


def _mlp(g0, g1, g2, W1, b1, W2, b2):
    gc = W2.shape[0]
    return pl.pallas_call(
        _mlp_step,
        out_shape=jax.ShapeDtypeStruct((NUM_GRAPHS, gc), jnp.float32),
    )(g0, g1, g2, W1[:_CH], W1[_CH:2 * _CH], W1[2 * _CH:], b1.reshape(1, gc),
      W2, b2.reshape(1, gc))


@jax.jit
def kernel(x_operation, batch_operation, tok_operation, Wl_operation,
           bl_operation, Wr_operation, br_operation, att_operation,
           bias_operation, x_machine, batch_machine, tok_machine, Wl_machine,
           bl_machine, Wr_machine, br_machine, att_machine, bias_machine,
           x_AGV, batch_AGV, tok_AGV, Wl_AGV, bl_AGV, Wr_AGV, br_AGV, att_AGV,
           bias_AGV, W1, b1, W2, b2):
    g_op = _gat_global(x_operation, batch_operation, tok_operation,
                       Wl_operation, bl_operation, Wr_operation, br_operation,
                       att_operation, bias_operation, tile=2000)
    g_ma = _gat_global(x_machine, batch_machine, tok_machine, Wl_machine,
                       bl_machine, Wr_machine, br_machine, att_machine,
                       bias_machine, tile=2000)
    g_ag = _gat_global(x_AGV, batch_AGV, tok_AGV, Wl_AGV, bl_AGV, Wr_AGV,
                       br_AGV, att_AGV, bias_AGV, tile=2000)
    graph_feature = _mlp(g_op, g_ma, g_ag, W1, b1, W2, b2)
    return (g_op, g_ma, g_ag, graph_feature)


# speedup vs baseline: 11.9832x; 1.1439x over previous
import functools

import jax
import jax.numpy as jnp
from jax.experimental import pallas as pl
from jax.experimental.pallas import tpu as pltpu

NUM_GRAPHS = 512
_CH = 256


def _gat_step(x_ref, b_ref, tok_ref, Wl_ref, bl_ref, Wr_ref, br_ref, att_ref,
              bias_ref, out_ref, acc_ref, den_ref, *, num_steps, tile):
    i = pl.program_id(0)

    @pl.when(i == 0)
    def _init():
        acc_ref[...] = jnp.zeros_like(acc_ref)
        den_ref[...] = jnp.zeros_like(den_ref)

    x = x_ref[...]
    xl = jnp.dot(x, Wl_ref[...],
                 preferred_element_type=jnp.float32) + bl_ref[...]
    r = jnp.dot(tok_ref[...], Wr_ref[...],
                preferred_element_type=jnp.float32) + br_ref[...]
    e = xl + r
    e = jnp.where(e >= 0.0, e, 0.2 * e)
    logit = jnp.sum(e * att_ref[...], axis=1, keepdims=True)
    ex = jnp.exp(logit)

    b = b_ref[...]
    onehot = (b == jax.lax.broadcasted_iota(jnp.int32, (tile, NUM_GRAPHS), 1))
    ow = onehot.astype(jnp.float32) * ex

    acc_ref[...] += jax.lax.dot_general(
        ow.astype(jnp.bfloat16), xl.astype(jnp.bfloat16),
        (((0,), (0,)), ((), ())),
        preferred_element_type=jnp.float32)
    den_ref[...] += jnp.sum(ow, axis=0, keepdims=True)

    @pl.when(i == num_steps - 1)
    def _finalize():
        den_col = jax.lax.transpose(den_ref[...], (1, 0))
        out_ref[...] = acc_ref[...] / (den_col + 1e-16) + bias_ref[...]


def _gat_global(x, batch, tok, Wl, bl, Wr, br, att, bias, *, tile):
    n = x.shape[0]
    assert n % tile == 0, (n, tile)
    num_steps = n // tile
    b2 = batch.reshape(n, 1)
    return pl.pallas_call(
        functools.partial(_gat_step, num_steps=num_steps, tile=tile),
        grid=(num_steps,),
        in_specs=[
            pl.BlockSpec((tile, _CH), lambda i: (i, 0)),
            pl.BlockSpec((tile, 1), lambda i: (i, 0)),
            pl.BlockSpec((1, _CH), lambda i: (0, 0)),
            pl.BlockSpec((_CH, _CH), lambda i: (0, 0)),
            pl.BlockSpec((1, _CH), lambda i: (0, 0)),
            pl.BlockSpec((_CH, _CH), lambda i: (0, 0)),
            pl.BlockSpec((1, _CH), lambda i: (0, 0)),
            pl.BlockSpec((1, _CH), lambda i: (0, 0)),
            pl.BlockSpec((1, _CH), lambda i: (0, 0)),
        ],
        out_specs=pl.BlockSpec((NUM_GRAPHS, _CH), lambda i: (0, 0)),
        out_shape=jax.ShapeDtypeStruct((NUM_GRAPHS, _CH), jnp.float32),
        scratch_shapes=[
            pltpu.VMEM((NUM_GRAPHS, _CH), jnp.float32),
            pltpu.VMEM((1, NUM_GRAPHS), jnp.float32),
        ],
        compiler_params=pltpu.CompilerParams(
            dimension_semantics=("arbitrary",)),
    )(x, b2, tok.reshape(1, _CH), Wl, bl.reshape(1, _CH), Wr,
      br.reshape(1, _CH), att.reshape(1, _CH), bias.reshape(1, _CH))


def _mlp_step(g0_ref, g1_ref, g2_ref, W1a_ref, W1b_ref, W1c_ref, b1_ref,
              W2_ref, b2_ref, out_ref):
    h = (jnp.dot(g0_ref[...], W1a_ref[...], preferred_element_type=jnp.float32)
         + jnp.dot(g1_ref[...], W1b_ref[...], preferred_element_type=jnp.float32)
         + jnp.dot(g2_ref[...], W1c_ref[...], preferred_element_type=jnp.float32)
         + b1_ref[...])
    h = jnp.where(h >= 0.0, h, 0.01 * h)
    out_ref[...] = jnp.dot(h, W2_ref[...],
                           preferred_element_type=jnp.float32) + b2_ref[...]


def _mlp(g0, g1, g2, W1, b1, W2, b2):
    gc = W2.shape[0]
    return pl.pallas_call(
        _mlp_step,
        out_shape=jax.ShapeDtypeStruct((NUM_GRAPHS, gc), jnp.float32),
    )(g0, g1, g2, W1[:_CH], W1[_CH:2 * _CH], W1[2 * _CH:], b1.reshape(1, gc),
      W2, b2.reshape(1, gc))


@jax.jit
def kernel(x_operation, batch_operation, tok_operation, Wl_operation,
           bl_operation, Wr_operation, br_operation, att_operation,
           bias_operation, x_machine, batch_machine, tok_machine, Wl_machine,
           bl_machine, Wr_machine, br_machine, att_machine, bias_machine,
           x_AGV, batch_AGV, tok_AGV, Wl_AGV, bl_AGV, Wr_AGV, br_AGV, att_AGV,
           bias_AGV, W1, b1, W2, b2):
    g_op = _gat_global(x_operation, batch_operation, tok_operation,
                       Wl_operation, bl_operation, Wr_operation, br_operation,
                       att_operation, bias_operation, tile=2000)
    g_ma = _gat_global(x_machine, batch_machine, tok_machine, Wl_machine,
                       bl_machine, Wr_machine, br_machine, att_machine,
                       bias_machine, tile=2000)
    g_ag = _gat_global(x_AGV, batch_AGV, tok_AGV, Wl_AGV, bl_AGV, Wr_AGV,
                       br_AGV, att_AGV, bias_AGV, tile=2000)
    graph_feature = _mlp(g_op, g_ma, g_ag, W1, b1, W2, b2)
    return (g_op, g_ma, g_ag, graph_feature)


# speedup vs baseline: 12.0169x; 1.0028x over previous
import functools

import jax
import jax.numpy as jnp
from jax.experimental import pallas as pl
from jax.experimental.pallas import tpu as pltpu

NUM_GRAPHS = 512
_CH = 256


def _gat_step(x_ref, b_ref, tok_ref, Wl_ref, bl_ref, Wr_ref, br_ref, att_ref,
              bias_ref, out_ref, acc_ref, den_ref, *, num_steps, tile):
    i = pl.program_id(0)

    @pl.when(i == 0)
    def _init():
        acc_ref[...] = jnp.zeros_like(acc_ref)
        den_ref[...] = jnp.zeros_like(den_ref)

    x = x_ref[...]
    xl = jnp.dot(x.astype(jnp.bfloat16), Wl_ref[...].astype(jnp.bfloat16),
                 preferred_element_type=jnp.float32) + bl_ref[...]
    r = jnp.dot(tok_ref[...], Wr_ref[...],
                preferred_element_type=jnp.float32) + br_ref[...]
    e = xl + r
    e = jnp.where(e >= 0.0, e, 0.2 * e)
    logit = jnp.sum(e * att_ref[...], axis=1, keepdims=True)
    ex = jnp.exp(logit)

    b = b_ref[...]
    onehot = (b == jax.lax.broadcasted_iota(jnp.int32, (tile, NUM_GRAPHS), 1))
    ow = onehot.astype(jnp.float32) * ex

    acc_ref[...] += jax.lax.dot_general(
        ow.astype(jnp.bfloat16), xl.astype(jnp.bfloat16),
        (((0,), (0,)), ((), ())),
        preferred_element_type=jnp.float32)
    den_ref[...] += jnp.sum(ow, axis=0, keepdims=True)

    @pl.when(i == num_steps - 1)
    def _finalize():
        den_col = jax.lax.transpose(den_ref[...], (1, 0))
        out_ref[...] = acc_ref[...] / (den_col + 1e-16) + bias_ref[...]


def _gat_global(x, batch, tok, Wl, bl, Wr, br, att, bias, *, tile):
    n = x.shape[0]
    assert n % tile == 0, (n, tile)
    num_steps = n // tile
    b2 = batch.reshape(n, 1)
    return pl.pallas_call(
        functools.partial(_gat_step, num_steps=num_steps, tile=tile),
        grid=(num_steps,),
        in_specs=[
            pl.BlockSpec((tile, _CH), lambda i: (i, 0)),
            pl.BlockSpec((tile, 1), lambda i: (i, 0)),
            pl.BlockSpec((1, _CH), lambda i: (0, 0)),
            pl.BlockSpec((_CH, _CH), lambda i: (0, 0)),
            pl.BlockSpec((1, _CH), lambda i: (0, 0)),
            pl.BlockSpec((_CH, _CH), lambda i: (0, 0)),
            pl.BlockSpec((1, _CH), lambda i: (0, 0)),
            pl.BlockSpec((1, _CH), lambda i: (0, 0)),
            pl.BlockSpec((1, _CH), lambda i: (0, 0)),
        ],
        out_specs=pl.BlockSpec((NUM_GRAPHS, _CH), lambda i: (0, 0)),
        out_shape=jax.ShapeDtypeStruct((NUM_GRAPHS, _CH), jnp.float32),
        scratch_shapes=[
            pltpu.VMEM((NUM_GRAPHS, _CH), jnp.float32),
            pltpu.VMEM((1, NUM_GRAPHS), jnp.float32),
        ],
        compiler_params=pltpu.CompilerParams(
            dimension_semantics=("arbitrary",)),
    )(x, b2, tok.reshape(1, _CH), Wl, bl.reshape(1, _CH), Wr,
      br.reshape(1, _CH), att.reshape(1, _CH), bias.reshape(1, _CH))


def _mlp_step(g0_ref, g1_ref, g2_ref, W1a_ref, W1b_ref, W1c_ref, b1_ref,
              W2_ref, b2_ref, out_ref):
    h = (jnp.dot(g0_ref[...], W1a_ref[...], preferred_element_type=jnp.float32)
         + jnp.dot(g1_ref[...], W1b_ref[...], preferred_element_type=jnp.float32)
         + jnp.dot(g2_ref[...], W1c_ref[...], preferred_element_type=jnp.float32)
         + b1_ref[...])
    h = jnp.where(h >= 0.0, h, 0.01 * h)
    out_ref[...] = jnp.dot(h, W2_ref[...],
                           preferred_element_type=jnp.float32) + b2_ref[...]


def _mlp(g0, g1, g2, W1, b1, W2, b2):
    gc = W2.shape[0]
    return pl.pallas_call(
        _mlp_step,
        out_shape=jax.ShapeDtypeStruct((NUM_GRAPHS, gc), jnp.float32),
    )(g0, g1, g2, W1[:_CH], W1[_CH:2 * _CH], W1[2 * _CH:], b1.reshape(1, gc),
      W2, b2.reshape(1, gc))


@jax.jit
def kernel(x_operation, batch_operation, tok_operation, Wl_operation,
           bl_operation, Wr_operation, br_operation, att_operation,
           bias_operation, x_machine, batch_machine, tok_machine, Wl_machine,
           bl_machine, Wr_machine, br_machine, att_machine, bias_machine,
           x_AGV, batch_AGV, tok_AGV, Wl_AGV, bl_AGV, Wr_AGV, br_AGV, att_AGV,
           bias_AGV, W1, b1, W2, b2):
    g_op = _gat_global(x_operation, batch_operation, tok_operation,
                       Wl_operation, bl_operation, Wr_operation, br_operation,
                       att_operation, bias_operation, tile=2000)
    g_ma = _gat_global(x_machine, batch_machine, tok_machine, Wl_machine,
                       bl_machine, Wr_machine, br_machine, att_machine,
                       bias_machine, tile=2000)
    g_ag = _gat_global(x_AGV, batch_AGV, tok_AGV, Wl_AGV, bl_AGV, Wr_AGV,
                       br_AGV, att_AGV, bias_AGV, tile=2000)
    graph_feature = _mlp(g_op, g_ma, g_ag, W1, b1, W2, b2)
    return (g_op, g_ma, g_ag, graph_feature)


# speedup vs baseline: 13.5646x; 1.1288x over previous
import functools

import jax
import jax.numpy as jnp
from jax.experimental import pallas as pl
from jax.experimental.pallas import tpu as pltpu

NUM_GRAPHS = 512
_CH = 256


def _gat_step(x_ref, b_ref, tok_ref, Wl_ref, bl_ref, Wr_ref, br_ref, att_ref,
              bias_ref, out_ref, acc_ref, *, num_steps, tile):
    i = pl.program_id(0)

    @pl.when(i == 0)
    def _init():
        acc_ref[...] = jnp.zeros_like(acc_ref)

    x = x_ref[...]
    xl = jnp.dot(x.astype(jnp.bfloat16), Wl_ref[...].astype(jnp.bfloat16),
                 preferred_element_type=jnp.float32) + bl_ref[...]
    r = jnp.dot(tok_ref[...], Wr_ref[...],
                preferred_element_type=jnp.float32) + br_ref[...]
    z = xl + r
    att = att_ref[...]
    att_sel = jnp.where(z >= 0.0, att, 0.2 * att)
    logit = jnp.sum(z * att_sel, axis=1, keepdims=True)
    ex = jnp.exp(logit)

    b16 = b_ref[...].astype(jnp.int16)
    onehot = (b16 == jax.lax.broadcasted_iota(jnp.int16, (tile, NUM_GRAPHS), 1))
    ow = jnp.where(onehot, ex.astype(jnp.bfloat16), jnp.bfloat16(0.0))

    xlp = jnp.concatenate(
        [xl.astype(jnp.bfloat16), jnp.ones((tile, 1), jnp.bfloat16)], axis=1)
    acc_ref[...] += jax.lax.dot_general(
        ow, xlp, (((0,), (0,)), ((), ())),
        preferred_element_type=jnp.float32)

    @pl.when(i == num_steps - 1)
    def _finalize():
        den_col = acc_ref[:, _CH:_CH + 1]
        out_ref[...] = acc_ref[:, :_CH] / (den_col + 1e-16) + bias_ref[...]


def _gat_global(x, batch, tok, Wl, bl, Wr, br, att, bias, *, tile):
    n = x.shape[0]
    assert n % tile == 0, (n, tile)
    num_steps = n // tile
    b2 = batch.reshape(n, 1)
    return pl.pallas_call(
        functools.partial(_gat_step, num_steps=num_steps, tile=tile),
        grid=(num_steps,),
        in_specs=[
            pl.BlockSpec((tile, _CH), lambda i: (i, 0)),
            pl.BlockSpec((tile, 1), lambda i: (i, 0)),
            pl.BlockSpec((1, _CH), lambda i: (0, 0)),
            pl.BlockSpec((_CH, _CH), lambda i: (0, 0)),
            pl.BlockSpec((1, _CH), lambda i: (0, 0)),
            pl.BlockSpec((_CH, _CH), lambda i: (0, 0)),
            pl.BlockSpec((1, _CH), lambda i: (0, 0)),
            pl.BlockSpec((1, _CH), lambda i: (0, 0)),
            pl.BlockSpec((1, _CH), lambda i: (0, 0)),
        ],
        out_specs=pl.BlockSpec((NUM_GRAPHS, _CH), lambda i: (0, 0)),
        out_shape=jax.ShapeDtypeStruct((NUM_GRAPHS, _CH), jnp.float32),
        scratch_shapes=[
            pltpu.VMEM((NUM_GRAPHS, _CH + 1), jnp.float32),
        ],
        compiler_params=pltpu.CompilerParams(
            dimension_semantics=("arbitrary",)),
    )(x, b2, tok.reshape(1, _CH), Wl, bl.reshape(1, _CH), Wr,
      br.reshape(1, _CH), att.reshape(1, _CH), bias.reshape(1, _CH))


def _mlp_step(g0_ref, g1_ref, g2_ref, W1a_ref, W1b_ref, W1c_ref, b1_ref,
              W2_ref, b2_ref, out_ref):
    h = (jnp.dot(g0_ref[...], W1a_ref[...], preferred_element_type=jnp.float32)
         + jnp.dot(g1_ref[...], W1b_ref[...], preferred_element_type=jnp.float32)
         + jnp.dot(g2_ref[...], W1c_ref[...], preferred_element_type=jnp.float32)
         + b1_ref[...])
    h = jnp.where(h >= 0.0, h, 0.01 * h)
    out_ref[...] = jnp.dot(h, W2_ref[...],
                           preferred_element_type=jnp.float32) + b2_ref[...]


def _mlp(g0, g1, g2, W1, b1, W2, b2):
    gc = W2.shape[0]
    return pl.pallas_call(
        _mlp_step,
        out_shape=jax.ShapeDtypeStruct((NUM_GRAPHS, gc), jnp.float32),
    )(g0, g1, g2, W1[:_CH], W1[_CH:2 * _CH], W1[2 * _CH:], b1.reshape(1, gc),
      W2, b2.reshape(1, gc))


@jax.jit
def kernel(x_operation, batch_operation, tok_operation, Wl_operation,
           bl_operation, Wr_operation, br_operation, att_operation,
           bias_operation, x_machine, batch_machine, tok_machine, Wl_machine,
           bl_machine, Wr_machine, br_machine, att_machine, bias_machine,
           x_AGV, batch_AGV, tok_AGV, Wl_AGV, bl_AGV, Wr_AGV, br_AGV, att_AGV,
           bias_AGV, W1, b1, W2, b2):
    g_op = _gat_global(x_operation, batch_operation, tok_operation,
                       Wl_operation, bl_operation, Wr_operation, br_operation,
                       att_operation, bias_operation, tile=2000)
    g_ma = _gat_global(x_machine, batch_machine, tok_machine, Wl_machine,
                       bl_machine, Wr_machine, br_machine, att_machine,
                       bias_machine, tile=2000)
    g_ag = _gat_global(x_AGV, batch_AGV, tok_AGV, Wl_AGV, bl_AGV, Wr_AGV,
                       br_AGV, att_AGV, bias_AGV, tile=2000)
    graph_feature = _mlp(g_op, g_ma, g_ag, W1, b1, W2, b2)
    return (g_op, g_ma, g_ag, graph_feature)
